# baseline (speedup 1.0000x reference)
import functools

import jax
import jax.numpy as jnp
from jax import lax
from jax.experimental import pallas as pl
from jax.experimental.pallas import tpu as pltpu
from jax.experimental.pallas import tpu_sc as plsc

B = 16384
D = 32
NC = 2
NS = 16
NW = NC * NS
B_PER_W = B // NW

@functools.cache
def _make_sc_gather():
    mesh = plsc.VectorSubcoreMesh(
        core_axis_name="c", subcore_axis_name="s",
        num_cores=NC, num_subcores=NS)

    @functools.partial(
        pl.kernel,
        out_type=[
            jax.ShapeDtypeStruct((B, D), jnp.float32),
            jax.ShapeDtypeStruct((B, D), jnp.float32),
        ],
        mesh=mesh,
        scratch_types=[
            pltpu.VMEM((B_PER_W,), jnp.int32),
            pltpu.VMEM((B_PER_W,), jnp.int32),
            pltpu.VMEM((B_PER_W, D), jnp.float32),
            pltpu.VMEM((B_PER_W, D), jnp.float32),
            pltpu.SemaphoreType.DMA,
            pltpu.SemaphoreType.DMA,
        ],
        compiler_params=pltpu.CompilerParams(use_tc_tiling_on_sc=False),
    )
    def sc_gather(user_table, item_table, uid, pid, out_u, out_i,
                  uidx_v, iidx_v, urows_v, irows_v, sem_u, sem_i):
        wid = lax.axis_index("s") * NC + lax.axis_index("c")
        base = wid * B_PER_W
        pltpu.sync_copy(uid.at[pl.ds(base, B_PER_W)], uidx_v)
        pltpu.sync_copy(pid.at[pl.ds(base, B_PER_W)], iidx_v)
        cu = pltpu.async_copy(user_table.at[uidx_v], urows_v, sem_u)
        ci = pltpu.async_copy(item_table.at[iidx_v], irows_v, sem_i)
        cu.wait()
        ci.wait()
        pltpu.sync_copy(urows_v, out_u.at[pl.ds(base, B_PER_W)])
        pltpu.sync_copy(irows_v, out_i.at[pl.ds(base, B_PER_W)])

    return sc_gather


def _mlp_body(ue_ref, ie_ref, w1u_ref, w1i_ref, b1_ref, w2_ref, b2_ref,
              w3_ref, b3_ref, out_ref):
    h = (
        jnp.dot(ue_ref[...], w1u_ref[...], preferred_element_type=jnp.float32)
        + jnp.dot(ie_ref[...], w1i_ref[...], preferred_element_type=jnp.float32)
        + b1_ref[...]
    )
    h = jnp.maximum(h, 0.0)
    h = jnp.dot(h, w2_ref[...], preferred_element_type=jnp.float32) + b2_ref[...]
    h = jnp.maximum(h, 0.0)
    out_ref[...] = (
        jnp.dot(h, w3_ref[...], preferred_element_type=jnp.float32) + b3_ref[...]
    )


def _mlp(ue, ie, w1u, w1i, b1, w2, b2, w3, b3, block_b=2048):
    grid = (B // block_b,)
    full = lambda shape: pl.BlockSpec(shape, lambda i: (0, 0))
    return pl.pallas_call(
        _mlp_body,
        grid=grid,
        in_specs=[
            pl.BlockSpec((block_b, D), lambda i: (i, 0)),
            pl.BlockSpec((block_b, D), lambda i: (i, 0)),
            full((D, 64)),
            full((D, 64)),
            full((1, 64)),
            full((64, 32)),
            full((1, 32)),
            full((32, 1)),
            full((1, 1)),
        ],
        out_specs=pl.BlockSpec((block_b, 1), lambda i: (i, 0)),
        out_shape=jax.ShapeDtypeStruct((B, 1), jnp.float32),
    )(ue, ie, w1u, w1i, b1, w2, b2, w3, b3)


def kernel(user_id, product_id, user_table, item_table, W1, b1, gamma, beta,
           moving_mean, moving_var, W2, b2, W3, b3):
    uid = user_id.astype(jnp.int32)
    pid = product_id.astype(jnp.int32)
    ue, ie = _make_sc_gather()(user_table, item_table, uid, pid)

    s = gamma * jax.lax.rsqrt(moving_var + 1e-3)
    t = beta - moving_mean * s
    w2f = W2 * s[:, None]
    b2f = b2 + t @ W2

    return _mlp(
        ue, ie,
        W1[:D], W1[D:], b1[None, :],
        w2f, b2f[None, :],
        W3, b3[None, :],
    )
